# baseline (speedup 1.0000x reference)
import functools

import jax
import jax.numpy as jnp
from jax import lax
from jax.experimental import pallas as pl
from jax.experimental.pallas import tpu as pltpu
from jax.experimental.pallas import tpu_sc as plsc

CBLK = 128
NBUF = 2


def _gather_kernel(s_len, b_len, v2, d, n_workers, num_cores):
    d2 = 2 * d
    mesh = plsc.VectorSubcoreMesh(core_axis_name="c", subcore_axis_name="s")

    @functools.partial(
        pl.kernel,
        mesh=mesh,
        compiler_params=pltpu.CompilerParams(needs_layout_passes=False),
        out_type=jax.ShapeDtypeStruct((s_len, d, b_len), jnp.float32),
        scratch_types=[
            pltpu.VMEM((s_len, CBLK), jnp.int32),
            pltpu.VMEM((NBUF, CBLK), jnp.int32),
            pltpu.VMEM((NBUF, CBLK, d2 + 5), jnp.float32),
            pltpu.VMEM((NBUF, 1, d, CBLK + 4), jnp.float32),
            pltpu.SemaphoreType.DMA,
            pltpu.SemaphoreType.DMA,
            pltpu.SemaphoreType.DMA,
            pltpu.SemaphoreType.DMA,
        ],
    )
    def k(ids_hbm, tab_hbm, out_hbm, ids_v, g_v, rows_v, slab_v,
          gsem0, gsem1, osem0, osem1):
        wid = lax.axis_index("s") * num_cores + lax.axis_index("c")
        base = wid * CBLK
        gsems = (gsem0, gsem1)
        osems = (osem0, osem1)

        pltpu.sync_copy(ids_hbm.at[:, pl.ds(base, CBLK)], ids_v)

        def fire_gather(s, b):
            for kk in range(CBLK // 16):
                ids16 = ids_v[s, pl.ds(16 * kk, 16)]
                g_v[b, pl.ds(16 * kk, 16)] = lax.shift_right_logical(ids16, 1)
            pltpu.async_copy(
                tab_hbm.at[g_v.at[b]], rows_v.at[b, :, pl.ds(0, d2)], gsems[b]
            )

        def wait_gather(b):
            pltpu.make_async_copy(
                tab_hbm.at[g_v.at[b]], rows_v.at[b, :, pl.ds(0, d2)], gsems[b]
            ).wait()

        def wait_out(s, b):
            pltpu.make_async_copy(
                slab_v.at[b, :, :, pl.ds(0, CBLK)],
                out_hbm.at[pl.ds(s, 1), :, pl.ds(base, CBLK)],
                osems[b],
            ).wait()

        fire_gather(0, 0)

        def step(s, b):
            @pl.when(s + 1 < s_len)
            def _():
                fire_gather(s + 1, 1 - b)

            wait_gather(b)

            @pl.when(s >= 2)
            def _():
                wait_out(s - 2, b)

            slab2 = slab_v.at[b, 0]
            cols = []
            for jb in range(CBLK // 16):
                ids16 = ids_v[s, pl.ds(16 * jb, 16)]
                cols.append(lax.shift_left(lax.rem(ids16, 2), 6))

            def ext_body(di, col_carry):
                for jb in range(CBLK // 16):
                    jv = lax.iota(jnp.int32, 16) + 16 * jb
                    val = plsc.load_gather(
                        rows_v.at[b], [jv, col_carry[jb] + di]
                    )
                    slab2[di, pl.ds(16 * jb, 16)] = val
                return col_carry

            lax.fori_loop(0, d, ext_body, tuple(cols))

            pltpu.async_copy(
                slab_v.at[b, :, :, pl.ds(0, CBLK)],
                out_hbm.at[pl.ds(s, 1), :, pl.ds(base, CBLK)],
                osems[b],
            )

        def body(g, carry):
            step(2 * g, 0)
            step(2 * g + 1, 1)
            return carry

        lax.fori_loop(0, s_len // 2, body, 0)

        wait_out(s_len - 2, (s_len - 2) % 2)
        wait_out(s_len - 1, (s_len - 1) % 2)

    return k


def kernel(token_ids, embed_weight):
    bt, s = token_ids.shape
    v, d = embed_weight.shape
    ids2 = token_ids.T.astype(jnp.int32)
    t2 = embed_weight.reshape(v // 2, 2 * d)
    info = plsc.get_sparse_core_info()
    n_workers = info.num_cores * info.num_subcores
    out = _gather_kernel(s, bt, v // 2, d, n_workers, info.num_cores)(ids2, t2)
    return out.transpose(2, 0, 1)


# speedup vs baseline: 1.8231x; 1.8231x over previous
import functools

import jax
import jax.numpy as jnp
from jax import lax
from jax.experimental import pallas as pl
from jax.experimental.pallas import tpu as pltpu
from jax.experimental.pallas import tpu_sc as plsc

CHUNK = 400
NBUF = 2


def _gather_kernel(n_rows, d, n_workers, num_cores):
    d2 = 2 * d
    b_per_w = n_rows // n_workers
    nchunks = b_per_w // CHUNK
    mesh = plsc.VectorSubcoreMesh(core_axis_name="c", subcore_axis_name="s")

    @functools.partial(
        pl.kernel,
        mesh=mesh,
        compiler_params=pltpu.CompilerParams(needs_layout_passes=False),
        out_type=jax.ShapeDtypeStruct((n_rows, d2), jnp.float32),
        scratch_types=[
            pltpu.VMEM((b_per_w,), jnp.int32),
            pltpu.VMEM((NBUF, CHUNK, d2), jnp.float32),
            pltpu.SemaphoreType.DMA,
            pltpu.SemaphoreType.DMA,
            pltpu.SemaphoreType.DMA,
            pltpu.SemaphoreType.DMA,
        ],
    )
    def k(ids_hbm, tab_hbm, out_hbm, idx_v, rows_v, gsem0, gsem1, osem0, osem1):
        wid = lax.axis_index("s") * num_cores + lax.axis_index("c")
        base = wid * b_per_w
        gsems = (gsem0, gsem1)
        osems = (osem0, osem1)

        pltpu.sync_copy(ids_hbm.at[pl.ds(base, b_per_w)], idx_v)

        def fire_gather(j, b):
            pltpu.async_copy(
                tab_hbm.at[idx_v.at[pl.ds(j * CHUNK, CHUNK)]],
                rows_v.at[b],
                gsems[b],
            )

        def wait_gather(b):
            pltpu.make_async_copy(
                tab_hbm.at[idx_v.at[pl.ds(0, CHUNK)]], rows_v.at[b], gsems[b]
            ).wait()

        def fire_out(j, b):
            pltpu.async_copy(
                rows_v.at[b],
                out_hbm.at[pl.ds(base + j * CHUNK, CHUNK)],
                osems[b],
            )

        def wait_out(b):
            pltpu.make_async_copy(
                rows_v.at[b],
                out_hbm.at[pl.ds(base, CHUNK)],
                osems[b],
            ).wait()

        fire_gather(0, 0)
        fire_gather(1, 1)

        def step(j, b):
            wait_gather(b)
            fire_out(j, b)
            jn = j + NBUF

            @pl.when(jn < nchunks)
            def _():
                wait_out(b)
                fire_gather(jn, b)

        def body(g, carry):
            step(2 * g, 0)
            step(2 * g + 1, 1)
            return carry

        lax.fori_loop(0, nchunks // 2, body, 0)
        wait_out(0)
        wait_out(1)

    return k


def kernel(token_ids, embed_weight):
    bt, s = token_ids.shape
    v, d = embed_weight.shape
    n = bt * s
    flat_ids = token_ids.reshape(n).astype(jnp.int32)
    t128 = jnp.pad(embed_weight, ((0, 0), (0, d)))
    info = plsc.get_sparse_core_info()
    n_workers = info.num_cores * info.num_subcores
    out = _gather_kernel(n, d, n_workers, info.num_cores)(flat_ids, t128)
    return out.reshape(bt, s, 2 * d)[:, :, :d]
